# baseline (speedup 1.0000x reference)
import functools

import jax
import jax.numpy as jnp
from jax import lax
from jax.experimental import pallas as pl
from jax.experimental.pallas import tpu as pltpu
from jax.experimental.pallas import tpu_sc as plsc

IMG_SIZE = 512.0


def kernel(y_pred, anchors):
    B, G, _, C = y_pred.shape
    A = anchors.shape[0]
    L = 16
    stride = IMG_SIZE / G
    x_t = jnp.transpose(y_pred, (0, 3, 1, 2))
    mul = jnp.broadcast_to(jnp.concatenate(
        [jnp.ones((A, 3), anchors.dtype), anchors], axis=1).reshape(C, 1),
        (C, 16))

    NW = 32
    BPW = B // NW
    mesh = plsc.VectorSubcoreMesh(core_axis_name="c", subcore_axis_name="s")

    @functools.partial(
        pl.kernel, mesh=mesh,
        out_type=jax.ShapeDtypeStruct((B, A, 5, G, G), jnp.float32),
        scratch_types=[
            pltpu.VMEM((G, G), jnp.float32),
            pltpu.VMEM((G, G), jnp.float32),
            pltpu.VMEM((C, 16), jnp.float32),
        ],
    )
    def k(x_hbm, mul_hbm, out_hbm, xin_v, r_v, mul_v):
        wid = lax.axis_index("s") * 2 + lax.axis_index("c")
        pltpu.sync_copy(mul_hbm, mul_v)
        for bb in range(BPW):
            b = wid * BPW + bb
            for c in range(C):
                a, f = c // 5, c % 5
                pltpu.sync_copy(x_hbm.at[b, c], xin_v)

                def body(i, _):
                    for j in range(G // L):
                        v = xin_v[i, pl.ds(j * L, L)]
                        e = jnp.exp(v)
                        if f < 3:
                            s = e / (1.0 + e)
                            if f == 0:
                                r = s
                            elif f == 1:
                                gx = (lax.iota(jnp.int32, L).astype(jnp.float32)
                                      + jnp.float32(j * L))
                                r = (s + gx) * stride
                            else:
                                gy = jnp.full((L,), i, jnp.float32)
                                r = (s + gy) * stride
                        else:
                            r = e * mul_v[c]
                        r_v[i, pl.ds(j * L, L)] = r
                    return 0

                lax.fori_loop(0, G, body, 0)
                pltpu.sync_copy(r_v, out_hbm.at[b, a, f])

    out = k(x_t, mul)
    return jnp.transpose(out, (0, 1, 3, 4, 2))


kernel = jax.jit(kernel)


# speedup vs baseline: 5.1478x; 5.1478x over previous
import functools

import jax
import jax.numpy as jnp
from jax.experimental import pallas as pl

IMG_SIZE = 512.0


def _decode_kernel(x_ref, mul_ref, o_ref, *, G, C, BB):
    stride = IMG_SIZE / G
    gx = jax.lax.broadcasted_iota(jnp.int32, (G, G), 1).astype(jnp.float32)
    gy = jax.lax.broadcasted_iota(jnp.int32, (G, G), 0).astype(jnp.float32)
    for bb in range(BB):
        for c in range(C):
            a, f = c // 5, c % 5
            v = x_ref[bb, c]
            if f < 3:
                s = jax.nn.sigmoid(v)
                if f == 0:
                    r = s
                elif f == 1:
                    r = (s + gx) * stride
                else:
                    r = (s + gy) * stride
            else:
                r = jnp.exp(v) * mul_ref[c]
            o_ref[bb, a, f] = r


@jax.jit
def kernel(y_pred, anchors):
    B, G, _, C = y_pred.shape
    A = anchors.shape[0]
    x_t = jnp.transpose(y_pred, (0, 3, 1, 2))
    mul = jnp.concatenate(
        [jnp.ones((A, 3), anchors.dtype), anchors], axis=1).reshape(C, 1, 1)
    BB = 16
    out = pl.pallas_call(
        functools.partial(_decode_kernel, G=G, C=C, BB=BB),
        grid=(B // BB,),
        in_specs=[
            pl.BlockSpec((BB, C, G, G), lambda b: (b, 0, 0, 0)),
            pl.BlockSpec((C, 1, 1), lambda b: (0, 0, 0)),
        ],
        out_specs=pl.BlockSpec((BB, A, 5, G, G), lambda b: (b, 0, 0, 0, 0)),
        out_shape=jax.ShapeDtypeStruct((B, A, 5, G, G), y_pred.dtype),
    )(x_t, mul)
    return jnp.transpose(out, (0, 1, 3, 4, 2))


# speedup vs baseline: 5.2373x; 1.0174x over previous
import functools

import jax
import jax.numpy as jnp
from jax.experimental import pallas as pl

IMG_SIZE = 512.0


def _decode_kernel(x_ref, anch_ref, o_ref, *, G, C, BB):
    stride = IMG_SIZE / G
    gx = jax.lax.broadcasted_iota(jnp.int32, (G, G), 1).astype(jnp.float32)
    gy = jax.lax.broadcasted_iota(jnp.int32, (G, G), 0).astype(jnp.float32)
    for bb in range(BB):
        for c in range(C):
            a, f = c // 5, c % 5
            v = x_ref[bb, c]
            if f < 3:
                s = jax.nn.sigmoid(v)
                if f == 0:
                    r = s
                elif f == 1:
                    r = (s + gx) * stride
                else:
                    r = (s + gy) * stride
            else:
                r = jnp.exp(v) * anch_ref[a, f - 3]
            o_ref[bb, a, f] = r


@jax.jit
def kernel(y_pred, anchors):
    B, G, _, C = y_pred.shape
    A = anchors.shape[0]
    x_t = jnp.transpose(y_pred, (0, 3, 1, 2))
    BB = 16
    out = pl.pallas_call(
        functools.partial(_decode_kernel, G=G, C=C, BB=BB),
        grid=(B // BB,),
        in_specs=[
            pl.BlockSpec((BB, C, G, G), lambda b: (b, 0, 0, 0)),
            pl.BlockSpec((A, 2), lambda b: (0, 0)),
        ],
        out_specs=pl.BlockSpec((BB, A, 5, G, G), lambda b: (b, 0, 0, 0, 0)),
        out_shape=jax.ShapeDtypeStruct((B, A, 5, G, G), y_pred.dtype),
    )(x_t, anchors)
    return jnp.transpose(out, (0, 1, 3, 4, 2))
